# baseline (speedup 1.0000x reference)
import functools

import jax
import jax.numpy as jnp
from jax import lax
from jax.experimental import pallas as pl
from jax.experimental.pallas import tpu as pltpu
from jax.experimental.pallas import tpu_sc as plsc


_S = 10000
_QL = 128
_UN = 25
_BIGI = 1 << 30


_ACC = 5


_GPS = 4


def _nn_body(q_ref, sx_ref, sy_ref, sz_ref, si_ref, o_ref):
    q = q_ref[...].reshape(_GPS, 3, _QL)
    qb = [
        [jnp.broadcast_to(q[g, c : c + 1], (8, _QL)) for c in range(3)]
        for g in range(_GPS)
    ]

    def chunk(i, carry):
        ds, js = [list(x) for x in carry[0]], [list(x) for x in carry[1]]
        for u in range(_UN):
            c = i * _UN + u
            a = u % _ACC
            base = pl.multiple_of(c * 8, 8)
            sx = sx_ref[pl.ds(base, 8), :]
            sy = sy_ref[pl.ds(base, 8), :]
            sz = sz_ref[pl.ds(base, 8), :]
            ii = si_ref[pl.ds(base, 8), :]
            for g in range(_GPS):
                dx = qb[g][0] - sx
                dy = qb[g][1] - sy
                dz = qb[g][2] - sz
                d = dx * dx + dy * dy + dz * dz
                m = d < ds[g][a]
                ds[g][a] = jnp.where(m, d, ds[g][a])
                js[g][a] = jnp.where(m, ii, js[g][a])
        return (tuple(tuple(x) for x in ds), tuple(tuple(x) for x in js))

    inf = jnp.full((8, _QL), jnp.inf, dtype=jnp.float32)
    zero = jnp.zeros((8, _QL), dtype=jnp.int32)
    carry0 = (((inf,) * _ACC,) * _GPS, ((zero,) * _ACC,) * _GPS)
    ds, js = lax.fori_loop(0, _S // (8 * _UN), chunk, carry0)
    rows = []
    for g in range(_GPS):
        bestd, besti = ds[g][0], js[g][0]
        for a in range(1, _ACC):
            m = (ds[g][a] < bestd) | ((ds[g][a] == bestd) & (js[g][a] < besti))
            bestd = jnp.where(m, ds[g][a], bestd)
            besti = jnp.where(m, js[g][a], besti)
        minv = jnp.min(bestd, axis=0, keepdims=True)
        cand = jnp.where(bestd == minv, besti, jnp.int32(_BIGI))
        rows.append(jnp.min(cand, axis=0, keepdims=True))
    o_ref[...] = jnp.concatenate(rows, axis=0).reshape(_GPS, 1, _QL)


def _nn_idx_tc(graph_pos, sub_pos):
    n = graph_pos.shape[0]
    ng = -(-n // _QL)
    ng += (-ng) % _GPS
    npad = ng * _QL
    q = jnp.pad(graph_pos, ((0, npad - n), (0, 0))).reshape(ng, _QL, 3)
    q = jnp.transpose(q, (0, 2, 1))
    sb = jnp.broadcast_to(sub_pos.T[:, :, None], (3, _S, _QL))
    si = jnp.broadcast_to(jnp.arange(_S, dtype=jnp.int32)[:, None], (_S, _QL))
    s_spec = pl.BlockSpec((_S, _QL), lambda i: (0, 0))
    idx = pl.pallas_call(
        _nn_body,
        grid=(ng // _GPS,),
        in_specs=[
            pl.BlockSpec((_GPS, 3, _QL), lambda i: (i, 0, 0)),
            s_spec, s_spec, s_spec, s_spec,
        ],
        out_specs=pl.BlockSpec((_GPS, 1, _QL), lambda i: (i, 0, 0)),
        out_shape=jax.ShapeDtypeStruct((ng, 1, _QL), jnp.int32),
    )(q, sb[0], sb[1], sb[2], si)
    return idx.reshape(-1)[:n]


_NC = 2
_NS = 16
_NW = _NC * _NS
_CH = 128


def _gather_sc(table, idx, n):
    mesh = plsc.VectorSubcoreMesh(core_axis_name="c", subcore_axis_name="s")
    kpw = -(-n // (_NW * _CH))
    kpw += 1 - (kpw % 2)
    bpad = _NW * kpw * _CH
    bpw = kpw * _CH
    idx_pad = jnp.pad(idx, (0, bpad - idx.shape[0]))

    @functools.partial(
        pl.kernel,
        mesh=mesh,
        out_type=jax.ShapeDtypeStruct((bpad, 128), jnp.float32),
        scratch_types=[
            pltpu.VMEM((bpw,), jnp.int32),
            pltpu.VMEM((_CH, 128), jnp.float32),
            pltpu.VMEM((_CH, 128), jnp.float32),
            pltpu.SemaphoreType.DMA,
            pltpu.SemaphoreType.DMA,
        ],
    )
    def k(idx_hbm, table_hbm, out_hbm, idx_v, buf0, buf1, sg0, sg1):
        wid = lax.axis_index("s") * _NC + lax.axis_index("c")
        base = pl.multiple_of(wid * bpw, _CH)
        pltpu.sync_copy(idx_hbm.at[pl.ds(base, bpw)], idx_v)

        def gather(c, buf, sem):
            off = pl.multiple_of(c * _CH, _CH)
            pltpu.async_copy(table_hbm.at[idx_v.at[pl.ds(off, _CH)]], buf, sem)

        def wait(buf, sem):
            pltpu.make_async_copy(table_hbm.at[pl.ds(0, _CH)], buf, sem).wait()

        def store(c, buf):
            off = pl.multiple_of(c * _CH, _CH)
            pltpu.sync_copy(buf, out_hbm.at[pl.ds(base + off, _CH)])

        gather(0, buf0, sg0)

        def pair(t, carry):
            c = t * 2
            gather(c + 1, buf1, sg1)
            wait(buf0, sg0)
            store(c, buf0)
            gather(c + 2, buf0, sg0)
            wait(buf1, sg1)
            store(c + 1, buf1)
            return carry

        lax.fori_loop(0, (kpw - 1) // 2, pair, 0)
        wait(buf0, sg0)
        store(kpw - 1, buf0)

    return k(idx_pad, table)[:n]


def kernel(subgraph_x, subgraph_pos, graph_pos):
    n = graph_pos.shape[0]
    idx = _nn_idx_tc(graph_pos, subgraph_pos)
    return _gather_sc(subgraph_x, idx, n)


# speedup vs baseline: 1.0388x; 1.0388x over previous
import functools

import jax
import jax.numpy as jnp
from jax import lax
from jax.experimental import pallas as pl
from jax.experimental.pallas import tpu as pltpu
from jax.experimental.pallas import tpu_sc as plsc


_S = 10000
_QL = 128
_UN = 50
_BIGI = 1 << 30


_ACC = 5


_GPS = 2


def _nn_body(q_ref, sx_ref, sy_ref, sz_ref, si_ref, o_ref):
    q = q_ref[...].reshape(_GPS, 3, _QL)
    qb = [
        [jnp.broadcast_to(q[g, c : c + 1], (8, _QL)) for c in range(3)]
        for g in range(_GPS)
    ]

    def chunk(i, carry):
        ds, js = [list(x) for x in carry[0]], [list(x) for x in carry[1]]
        for u in range(_UN):
            c = i * _UN + u
            a = u % _ACC
            base = pl.multiple_of(c * 8, 8)
            sx = sx_ref[pl.ds(base, 8), :]
            sy = sy_ref[pl.ds(base, 8), :]
            sz = sz_ref[pl.ds(base, 8), :]
            ii = si_ref[pl.ds(base, 8), :]
            for g in range(_GPS):
                dx = qb[g][0] - sx
                dy = qb[g][1] - sy
                dz = qb[g][2] - sz
                d = dx * dx + dy * dy + dz * dz
                m = d < ds[g][a]
                ds[g][a] = jnp.where(m, d, ds[g][a])
                js[g][a] = jnp.where(m, ii, js[g][a])
        return (tuple(tuple(x) for x in ds), tuple(tuple(x) for x in js))

    inf = jnp.full((8, _QL), jnp.inf, dtype=jnp.float32)
    zero = jnp.zeros((8, _QL), dtype=jnp.int32)
    carry0 = (((inf,) * _ACC,) * _GPS, ((zero,) * _ACC,) * _GPS)
    ds, js = lax.fori_loop(0, _S // (8 * _UN), chunk, carry0)
    rows = []
    for g in range(_GPS):
        bestd, besti = ds[g][0], js[g][0]
        for a in range(1, _ACC):
            m = (ds[g][a] < bestd) | ((ds[g][a] == bestd) & (js[g][a] < besti))
            bestd = jnp.where(m, ds[g][a], bestd)
            besti = jnp.where(m, js[g][a], besti)
        minv = jnp.min(bestd, axis=0, keepdims=True)
        cand = jnp.where(bestd == minv, besti, jnp.int32(_BIGI))
        rows.append(jnp.min(cand, axis=0, keepdims=True))
    o_ref[...] = jnp.concatenate(rows, axis=0).reshape(_GPS, 1, _QL)


def _nn_idx_tc(graph_pos, sub_pos):
    n = graph_pos.shape[0]
    ng = -(-n // _QL)
    ng += (-ng) % _GPS
    npad = ng * _QL
    q = jnp.pad(graph_pos, ((0, npad - n), (0, 0))).reshape(ng, _QL, 3)
    q = jnp.transpose(q, (0, 2, 1))
    sb = jnp.broadcast_to(sub_pos.T[:, :, None], (3, _S, _QL))
    si = jnp.broadcast_to(jnp.arange(_S, dtype=jnp.int32)[:, None], (_S, _QL))
    s_spec = pl.BlockSpec((_S, _QL), lambda i: (0, 0))
    idx = pl.pallas_call(
        _nn_body,
        grid=(ng // _GPS,),
        in_specs=[
            pl.BlockSpec((_GPS, 3, _QL), lambda i: (i, 0, 0)),
            s_spec, s_spec, s_spec, s_spec,
        ],
        out_specs=pl.BlockSpec((_GPS, 1, _QL), lambda i: (i, 0, 0)),
        out_shape=jax.ShapeDtypeStruct((ng, 1, _QL), jnp.int32),
    )(q, sb[0], sb[1], sb[2], si)
    return idx.reshape(-1)[:n]


_NC = 2
_NS = 16
_NW = _NC * _NS
_CH = 128


def _gather_sc(table, idx, n):
    mesh = plsc.VectorSubcoreMesh(core_axis_name="c", subcore_axis_name="s")
    kpw = -(-n // (_NW * _CH))
    kpw += 1 - (kpw % 2)
    bpad = _NW * kpw * _CH
    bpw = kpw * _CH
    idx_pad = jnp.pad(idx, (0, bpad - idx.shape[0]))

    @functools.partial(
        pl.kernel,
        mesh=mesh,
        out_type=jax.ShapeDtypeStruct((bpad, 128), jnp.float32),
        scratch_types=[
            pltpu.VMEM((bpw,), jnp.int32),
            pltpu.VMEM((_CH, 128), jnp.float32),
            pltpu.VMEM((_CH, 128), jnp.float32),
            pltpu.SemaphoreType.DMA,
            pltpu.SemaphoreType.DMA,
        ],
    )
    def k(idx_hbm, table_hbm, out_hbm, idx_v, buf0, buf1, sg0, sg1):
        wid = lax.axis_index("s") * _NC + lax.axis_index("c")
        base = pl.multiple_of(wid * bpw, _CH)
        pltpu.sync_copy(idx_hbm.at[pl.ds(base, bpw)], idx_v)

        def gather(c, buf, sem):
            off = pl.multiple_of(c * _CH, _CH)
            pltpu.async_copy(table_hbm.at[idx_v.at[pl.ds(off, _CH)]], buf, sem)

        def wait(buf, sem):
            pltpu.make_async_copy(table_hbm.at[pl.ds(0, _CH)], buf, sem).wait()

        def store(c, buf):
            off = pl.multiple_of(c * _CH, _CH)
            pltpu.sync_copy(buf, out_hbm.at[pl.ds(base + off, _CH)])

        gather(0, buf0, sg0)

        def pair(t, carry):
            c = t * 2
            gather(c + 1, buf1, sg1)
            wait(buf0, sg0)
            store(c, buf0)
            gather(c + 2, buf0, sg0)
            wait(buf1, sg1)
            store(c + 1, buf1)
            return carry

        lax.fori_loop(0, (kpw - 1) // 2, pair, 0)
        wait(buf0, sg0)
        store(kpw - 1, buf0)

    return k(idx_pad, table)[:n]


def kernel(subgraph_x, subgraph_pos, graph_pos):
    n = graph_pos.shape[0]
    idx = _nn_idx_tc(graph_pos, subgraph_pos)
    return _gather_sc(subgraph_x, idx, n)


# speedup vs baseline: 1.0628x; 1.0231x over previous
import functools

import jax
import jax.numpy as jnp
from jax import lax
from jax.experimental import pallas as pl
from jax.experimental.pallas import tpu as pltpu
from jax.experimental.pallas import tpu_sc as plsc


_S = 10000
_QL = 128
_UN = 125
_BIGI = 1 << 30


_ACC = 5


_GPS = 2


def _nn_body(q_ref, sx_ref, sy_ref, sz_ref, si_ref, o_ref):
    q = q_ref[...].reshape(_GPS, 3, _QL)
    qb = [
        [jnp.broadcast_to(q[g, c : c + 1], (8, _QL)) for c in range(3)]
        for g in range(_GPS)
    ]

    def chunk(i, carry):
        ds, js = [list(x) for x in carry[0]], [list(x) for x in carry[1]]
        for u in range(_UN):
            c = i * _UN + u
            a = u % _ACC
            base = pl.multiple_of(c * 8, 8)
            sx = sx_ref[pl.ds(base, 8), :]
            sy = sy_ref[pl.ds(base, 8), :]
            sz = sz_ref[pl.ds(base, 8), :]
            ii = si_ref[pl.ds(base, 8), :]
            for g in range(_GPS):
                dx = qb[g][0] - sx
                dy = qb[g][1] - sy
                dz = qb[g][2] - sz
                d = dx * dx + dy * dy + dz * dz
                m = d < ds[g][a]
                ds[g][a] = jnp.where(m, d, ds[g][a])
                js[g][a] = jnp.where(m, ii, js[g][a])
        return (tuple(tuple(x) for x in ds), tuple(tuple(x) for x in js))

    inf = jnp.full((8, _QL), jnp.inf, dtype=jnp.float32)
    zero = jnp.zeros((8, _QL), dtype=jnp.int32)
    carry0 = (((inf,) * _ACC,) * _GPS, ((zero,) * _ACC,) * _GPS)
    ds, js = lax.fori_loop(0, _S // (8 * _UN), chunk, carry0)
    rows = []
    for g in range(_GPS):
        bestd, besti = ds[g][0], js[g][0]
        for a in range(1, _ACC):
            m = (ds[g][a] < bestd) | ((ds[g][a] == bestd) & (js[g][a] < besti))
            bestd = jnp.where(m, ds[g][a], bestd)
            besti = jnp.where(m, js[g][a], besti)
        minv = jnp.min(bestd, axis=0, keepdims=True)
        cand = jnp.where(bestd == minv, besti, jnp.int32(_BIGI))
        rows.append(jnp.min(cand, axis=0, keepdims=True))
    o_ref[...] = jnp.concatenate(rows, axis=0).reshape(_GPS, 1, _QL)


def _nn_idx_tc(graph_pos, sub_pos):
    n = graph_pos.shape[0]
    ng = -(-n // _QL)
    ng += (-ng) % _GPS
    npad = ng * _QL
    q = jnp.pad(graph_pos, ((0, npad - n), (0, 0))).reshape(ng, _QL, 3)
    q = jnp.transpose(q, (0, 2, 1))
    sb = jnp.broadcast_to(sub_pos.T[:, :, None], (3, _S, _QL))
    si = jnp.broadcast_to(jnp.arange(_S, dtype=jnp.int32)[:, None], (_S, _QL))
    s_spec = pl.BlockSpec((_S, _QL), lambda i: (0, 0))
    idx = pl.pallas_call(
        _nn_body,
        grid=(ng // _GPS,),
        in_specs=[
            pl.BlockSpec((_GPS, 3, _QL), lambda i: (i, 0, 0)),
            s_spec, s_spec, s_spec, s_spec,
        ],
        out_specs=pl.BlockSpec((_GPS, 1, _QL), lambda i: (i, 0, 0)),
        out_shape=jax.ShapeDtypeStruct((ng, 1, _QL), jnp.int32),
    )(q, sb[0], sb[1], sb[2], si)
    return idx.reshape(-1)[:n]


_NC = 2
_NS = 16
_NW = _NC * _NS
_CH = 128


def _gather_sc(table, idx, n):
    mesh = plsc.VectorSubcoreMesh(core_axis_name="c", subcore_axis_name="s")
    kpw = -(-n // (_NW * _CH))
    kpw += 1 - (kpw % 2)
    bpad = _NW * kpw * _CH
    bpw = kpw * _CH
    idx_pad = jnp.pad(idx, (0, bpad - idx.shape[0]))

    @functools.partial(
        pl.kernel,
        mesh=mesh,
        out_type=jax.ShapeDtypeStruct((bpad, 128), jnp.float32),
        scratch_types=[
            pltpu.VMEM((bpw,), jnp.int32),
            pltpu.VMEM((_CH, 128), jnp.float32),
            pltpu.VMEM((_CH, 128), jnp.float32),
            pltpu.SemaphoreType.DMA,
            pltpu.SemaphoreType.DMA,
        ],
    )
    def k(idx_hbm, table_hbm, out_hbm, idx_v, buf0, buf1, sg0, sg1):
        wid = lax.axis_index("s") * _NC + lax.axis_index("c")
        base = pl.multiple_of(wid * bpw, _CH)
        pltpu.sync_copy(idx_hbm.at[pl.ds(base, bpw)], idx_v)

        def gather(c, buf, sem):
            off = pl.multiple_of(c * _CH, _CH)
            pltpu.async_copy(table_hbm.at[idx_v.at[pl.ds(off, _CH)]], buf, sem)

        def wait(buf, sem):
            pltpu.make_async_copy(table_hbm.at[pl.ds(0, _CH)], buf, sem).wait()

        def store(c, buf):
            off = pl.multiple_of(c * _CH, _CH)
            pltpu.sync_copy(buf, out_hbm.at[pl.ds(base + off, _CH)])

        gather(0, buf0, sg0)

        def pair(t, carry):
            c = t * 2
            gather(c + 1, buf1, sg1)
            wait(buf0, sg0)
            store(c, buf0)
            gather(c + 2, buf0, sg0)
            wait(buf1, sg1)
            store(c + 1, buf1)
            return carry

        lax.fori_loop(0, (kpw - 1) // 2, pair, 0)
        wait(buf0, sg0)
        store(kpw - 1, buf0)

    return k(idx_pad, table)[:n]


def kernel(subgraph_x, subgraph_pos, graph_pos):
    n = graph_pos.shape[0]
    idx = _nn_idx_tc(graph_pos, subgraph_pos)
    return _gather_sc(subgraph_x, idx, n)


# speedup vs baseline: 1.0708x; 1.0075x over previous
import functools

import jax
import jax.numpy as jnp
from jax import lax
from jax.experimental import pallas as pl
from jax.experimental.pallas import tpu as pltpu
from jax.experimental.pallas import tpu_sc as plsc


_S = 10000
_QL = 128
_UN = 250
_BIGI = 1 << 30


_ACC = 5


_GPS = 2


def _nn_body(q_ref, sx_ref, sy_ref, sz_ref, si_ref, o_ref):
    q = q_ref[...].reshape(_GPS, 3, _QL)
    qb = [
        [jnp.broadcast_to(q[g, c : c + 1], (8, _QL)) for c in range(3)]
        for g in range(_GPS)
    ]

    def chunk(i, carry):
        ds, js = [list(x) for x in carry[0]], [list(x) for x in carry[1]]
        for u in range(_UN):
            c = i * _UN + u
            a = u % _ACC
            base = pl.multiple_of(c * 8, 8)
            sx = sx_ref[pl.ds(base, 8), :]
            sy = sy_ref[pl.ds(base, 8), :]
            sz = sz_ref[pl.ds(base, 8), :]
            ii = si_ref[pl.ds(base, 8), :]
            for g in range(_GPS):
                dx = qb[g][0] - sx
                dy = qb[g][1] - sy
                dz = qb[g][2] - sz
                d = dx * dx + dy * dy + dz * dz
                m = d < ds[g][a]
                ds[g][a] = jnp.where(m, d, ds[g][a])
                js[g][a] = jnp.where(m, ii, js[g][a])
        return (tuple(tuple(x) for x in ds), tuple(tuple(x) for x in js))

    inf = jnp.full((8, _QL), jnp.inf, dtype=jnp.float32)
    zero = jnp.zeros((8, _QL), dtype=jnp.int32)
    carry0 = (((inf,) * _ACC,) * _GPS, ((zero,) * _ACC,) * _GPS)
    ds, js = lax.fori_loop(0, _S // (8 * _UN), chunk, carry0)
    rows = []
    for g in range(_GPS):
        bestd, besti = ds[g][0], js[g][0]
        for a in range(1, _ACC):
            m = (ds[g][a] < bestd) | ((ds[g][a] == bestd) & (js[g][a] < besti))
            bestd = jnp.where(m, ds[g][a], bestd)
            besti = jnp.where(m, js[g][a], besti)
        minv = jnp.min(bestd, axis=0, keepdims=True)
        cand = jnp.where(bestd == minv, besti, jnp.int32(_BIGI))
        rows.append(jnp.min(cand, axis=0, keepdims=True))
    o_ref[...] = jnp.concatenate(rows, axis=0).reshape(_GPS, 1, _QL)


def _nn_idx_tc(graph_pos, sub_pos):
    n = graph_pos.shape[0]
    ng = -(-n // _QL)
    ng += (-ng) % _GPS
    npad = ng * _QL
    q = jnp.pad(graph_pos, ((0, npad - n), (0, 0))).reshape(ng, _QL, 3)
    q = jnp.transpose(q, (0, 2, 1))
    sb = jnp.broadcast_to(sub_pos.T[:, :, None], (3, _S, _QL))
    si = jnp.broadcast_to(jnp.arange(_S, dtype=jnp.int32)[:, None], (_S, _QL))
    s_spec = pl.BlockSpec((_S, _QL), lambda i: (0, 0))
    idx = pl.pallas_call(
        _nn_body,
        grid=(ng // _GPS,),
        in_specs=[
            pl.BlockSpec((_GPS, 3, _QL), lambda i: (i, 0, 0)),
            s_spec, s_spec, s_spec, s_spec,
        ],
        out_specs=pl.BlockSpec((_GPS, 1, _QL), lambda i: (i, 0, 0)),
        out_shape=jax.ShapeDtypeStruct((ng, 1, _QL), jnp.int32),
    )(q, sb[0], sb[1], sb[2], si)
    return idx.reshape(-1)[:n]


_NC = 2
_NS = 16
_NW = _NC * _NS
_CH = 128


def _gather_sc(table, idx, n):
    mesh = plsc.VectorSubcoreMesh(core_axis_name="c", subcore_axis_name="s")
    kpw = -(-n // (_NW * _CH))
    kpw += 1 - (kpw % 2)
    bpad = _NW * kpw * _CH
    bpw = kpw * _CH
    idx_pad = jnp.pad(idx, (0, bpad - idx.shape[0]))

    @functools.partial(
        pl.kernel,
        mesh=mesh,
        out_type=jax.ShapeDtypeStruct((bpad, 128), jnp.float32),
        scratch_types=[
            pltpu.VMEM((bpw,), jnp.int32),
            pltpu.VMEM((_CH, 128), jnp.float32),
            pltpu.VMEM((_CH, 128), jnp.float32),
            pltpu.SemaphoreType.DMA,
            pltpu.SemaphoreType.DMA,
        ],
    )
    def k(idx_hbm, table_hbm, out_hbm, idx_v, buf0, buf1, sg0, sg1):
        wid = lax.axis_index("s") * _NC + lax.axis_index("c")
        base = pl.multiple_of(wid * bpw, _CH)
        pltpu.sync_copy(idx_hbm.at[pl.ds(base, bpw)], idx_v)

        def gather(c, buf, sem):
            off = pl.multiple_of(c * _CH, _CH)
            pltpu.async_copy(table_hbm.at[idx_v.at[pl.ds(off, _CH)]], buf, sem)

        def wait(buf, sem):
            pltpu.make_async_copy(table_hbm.at[pl.ds(0, _CH)], buf, sem).wait()

        def store(c, buf):
            off = pl.multiple_of(c * _CH, _CH)
            pltpu.sync_copy(buf, out_hbm.at[pl.ds(base + off, _CH)])

        gather(0, buf0, sg0)

        def pair(t, carry):
            c = t * 2
            gather(c + 1, buf1, sg1)
            wait(buf0, sg0)
            store(c, buf0)
            gather(c + 2, buf0, sg0)
            wait(buf1, sg1)
            store(c + 1, buf1)
            return carry

        lax.fori_loop(0, (kpw - 1) // 2, pair, 0)
        wait(buf0, sg0)
        store(kpw - 1, buf0)

    return k(idx_pad, table)[:n]


def kernel(subgraph_x, subgraph_pos, graph_pos):
    n = graph_pos.shape[0]
    idx = _nn_idx_tc(graph_pos, subgraph_pos)
    return _gather_sc(subgraph_x, idx, n)
